# Initial kernel scaffold; baseline (speedup 1.0000x reference)
import jax
import jax.numpy as jnp
from jax.experimental import pallas as pl


def kernel(x, W, adj_values, edge_index):
    raise NotImplementedError("write your pallas kernel here")


# speedup vs baseline: 3.4481x; 3.4481x over previous
import functools

import jax
import jax.numpy as jnp
from jax import lax
from jax.experimental import pallas as pl
from jax.experimental.pallas import tpu as pltpu
from jax.experimental.pallas import tpu_sc as plsc

N_NODES = 10000
D_IN = 256
D_OUT = 256
DH = 128
N_TILES = 16
CHUNK = 128
ROWS_PER_TILE = 624
TAIL_ROWS = N_NODES - N_TILES * ROWS_PER_TILE
DRAIN = 104


def _mm_body(x_ref, w_ref, o_ref):
    o_ref[...] = lax.dot_general(
        x_ref[...], w_ref[...],
        dimension_numbers=(((1,), (1,)), ((), ())),
        preferred_element_type=jnp.float32,
    )


def _matmul_halves(x, W):
    n = x.shape[0]
    blk = 1000
    return pl.pallas_call(
        _mm_body,
        grid=(2, n // blk),
        in_specs=[
            pl.BlockSpec((blk, D_IN), lambda c, i: (i, 0)),
            pl.BlockSpec((DH, D_IN), lambda c, i: (c, 0)),
        ],
        out_specs=pl.BlockSpec((blk, DH), lambda c, i: (c * (n // blk) + i, 0)),
        out_shape=jax.ShapeDtypeStruct((2 * n, DH), jnp.float32),
    )(x, W)


def _sc_aggregate(h2, srcs, dsts, vals, zrows, nch):
    mesh = plsc.VectorSubcoreMesh(core_axis_name="c", subcore_axis_name="s")

    @functools.partial(
        pl.kernel,
        mesh=mesh,
        out_type=jax.ShapeDtypeStruct((2, N_NODES, DH), jnp.float32),
        scratch_types=[
            pltpu.VMEM((nch, CHUNK), jnp.int32),
            pltpu.VMEM((nch, CHUNK), jnp.int32),
            pltpu.VMEM((nch, CHUNK), jnp.float32),
            pltpu.VMEM((CHUNK, DH), jnp.float32),
            pltpu.VMEM_SHARED((N_NODES, DH), jnp.float32),
            pltpu.SemaphoreType.DMA,
        ],
    )
    def body(h_ref, src_ref, dst_ref, val_ref, z_ref, out_ref,
             src_v, dst_v, val_v, rows_v, acc_s, sem):
        c = lax.axis_index("c")
        s = lax.axis_index("s")

        pltpu.sync_copy(src_ref.at[c, s], src_v)
        pltpu.sync_copy(dst_ref.at[s], dst_v)
        pltpu.sync_copy(val_ref.at[s], val_v)

        pltpu.sync_copy(z_ref, acc_s.at[pl.ds(s * ROWS_PER_TILE, ROWS_PER_TILE)])

        @pl.when(s == 0)
        def _():
            pltpu.sync_copy(
                z_ref.at[pl.ds(0, TAIL_ROWS)],
                acc_s.at[pl.ds(N_TILES * ROWS_PER_TILE, TAIL_ROWS)],
            )

        plsc.subcore_barrier()

        def chunk_body(j, carry):
            pltpu.async_copy(h_ref.at[src_v.at[j]], rows_v, sem).wait()

            def blk_body(b, carry2):
                vblk = val_v[j, pl.ds(b * 16, 16)]
                for k in range(16):
                    scal = vblk[k]
                    e = b * 16 + k
                    for f in range(DH // 16):
                        col = pl.ds(f * 16, 16)
                        rows_v[e, col] = rows_v[e, col] * scal
                return carry2

            lax.fori_loop(0, CHUNK // 16, blk_body, 0)

            pltpu.sync_copy(rows_v, acc_s.at[dst_v.at[j]], add=True)
            return carry

        lax.fori_loop(0, nch, chunk_body, 0)
        plsc.subcore_barrier()

        def drain_chunk(row0, nrows):
            sl = pl.ds(row0, nrows)
            pltpu.sync_copy(acc_s.at[sl], rows_v.at[pl.ds(0, nrows)])

            def relu_body(i, carry2):
                for f in range(DH // 16):
                    col = pl.ds(f * 16, 16)
                    rows_v[i, col] = jnp.maximum(rows_v[i, col], 0.0)
                return carry2

            lax.fori_loop(0, nrows, relu_body, 0)
            pltpu.sync_copy(rows_v.at[pl.ds(0, nrows)], out_ref.at[c, sl])

        base = s * ROWS_PER_TILE
        for k in range(ROWS_PER_TILE // DRAIN):
            drain_chunk(base + k * DRAIN, DRAIN)

        @pl.when(s == 0)
        def _():
            drain_chunk(N_TILES * ROWS_PER_TILE, TAIL_ROWS)

    return body(h2, srcs, dsts, vals, zrows)


def kernel(x, W, adj_values, edge_index):
    n, e = x.shape[0], adj_values.shape[0]
    nch = -(-e // (N_TILES * CHUNK))
    e_pad = nch * N_TILES * CHUNK
    pad = e_pad - e

    h2 = _matmul_halves(x, W)

    src = jnp.pad(edge_index[1], (0, pad))
    srcs = jnp.stack([src, src + n]).reshape(2, N_TILES, nch, CHUNK)
    dsts = jnp.pad(edge_index[0], (0, pad)).reshape(N_TILES, nch, CHUNK)
    vals = jnp.pad(adj_values, (0, pad)).reshape(N_TILES, nch, CHUNK)
    zrows = jnp.zeros((ROWS_PER_TILE, DH), jnp.float32)

    out2 = _sc_aggregate(h2, srcs, dsts, vals, zrows, nch)
    return out2.transpose(1, 0, 2).reshape(n, D_OUT)
